# baseline (speedup 1.0000x reference)
import functools

import jax
import jax.numpy as jnp
import numpy as np
from jax.experimental import pallas as pl
from jax.experimental.pallas import tpu as pltpu


def _lowpass_matrix(n, cutoff_ratio=0.25):
    r = max(int(n * cutoff_ratio), 1)
    shifted = (np.arange(n) + n // 2) % n
    mask = (np.abs(shifted - n // 2) <= r).astype(np.float64)
    eye = np.eye(n)
    B = np.fft.ifft(mask[:, None] * np.fft.fft(eye, axis=0), axis=0).real
    return jnp.asarray(B, jnp.float32)


def _sep_masks(H, W):
    yy, xx = np.meshgrid(np.arange(H), np.arange(W), indexing='ij')
    ym = np.zeros((8, H * W), np.float32)
    xm = np.zeros((8, H * W), np.float32)
    for i, dlt in enumerate(range(-2, 3)):
        ym[i] = ((yy + dlt >= 0) & (yy + dlt < H)).reshape(-1)
        xm[i] = ((xx + dlt >= 0) & (xx + dlt < W)).reshape(-1)
    return (jnp.asarray(ym, jnp.bfloat16), jnp.asarray(xm, jnp.bfloat16))


def _gbf_body(C, H, W,
              r_ref, d_ref, ym_ref, xm_ref,
              wsp_ref, bsp_ref, w1t_ref, b1_ref, w2_ref, we_ref,
              gs_ref, gb_ref, wc0_ref, wcs_ref, s5_ref, b5_ref, s3_ref, b3_ref,
              r_out_ref, s_out_ref, d_out_ref,
              dy_ref, tap_ref):
    HW = H * W
    wsp, bsp = wsp_ref[...], bsp_ref[...]
    w1t, b1 = w1t_ref[...], b1_ref[...]
    w2, we = w2_ref[...], we_ref[...]
    g_scale, g_bias = gs_ref[...], gb_ref[...]

    def glore(x16):
        y = jnp.dot(wsp, x16, preferred_element_type=jnp.float32) + bsp
        xs = y[:2 * C, :].astype(jnp.bfloat16)
        xp = y[2 * C:, :].astype(jnp.bfloat16)
        xn = jax.lax.dot_general(
            xs, xp, (((1,), (1,)), ((), ())),
            preferred_element_type=jnp.float32)
        h = jnp.dot(xn, w1t, preferred_element_type=jnp.float32) + b1
        h = jnp.maximum(h + xn, 0.0)
        xrel = jnp.dot(w2, h, preferred_element_type=jnp.float32)
        wx = jnp.dot(we, xrel, preferred_element_type=jnp.float32)
        ext = jnp.dot(wx.astype(jnp.bfloat16), xp,
                      preferred_element_type=jnp.float32)
        return x16.astype(jnp.float32) + ext * g_scale + g_bias

    r = glore(r_ref[0])
    d = glore(d_ref[0])
    s16 = (r + d).astype(jnp.bfloat16)

    for i, dy in enumerate(range(-2, 3)):
        shift = (-(dy * W)) % HW
        t = pltpu.roll(s16, shift, 1) if shift else s16
        if dy:
            t = t * ym_ref[pl.ds(i, 1), :]
        dy_ref[pl.ds(i * C, C), :] = t

    stack = dy_ref[...]
    for j, dx in enumerate((-2, -1, 1, 2)):
        t = pltpu.roll(stack, (-dx) % HW, 1) * xm_ref[pl.ds(dx + 2, 1), :]
        tap_ref[pl.ds(j * 5 * C, 5 * C), :] = t

    acc = (jnp.dot(wc0_ref[...], stack, preferred_element_type=jnp.float32)
           + jnp.dot(wcs_ref[...], tap_ref[...],
                     preferred_element_type=jnp.float32))
    t5 = jnp.maximum(acc[:C] * s5_ref[...] + b5_ref[...], 0.0)
    t3 = jnp.maximum(acc[C:] * s3_ref[...] + b3_ref[...], 0.0)
    s = t5 - t3

    r_out_ref[0] = s + r
    s_out_ref[0] = s
    d_out_ref[0] = d.astype(jnp.bfloat16)


def kernel(r, d, ws, bs, wp, bp, w1, b1, w2, we, g_scale, g_bias,
           w3, bn3_s, bn3_b, w5, bn5_s, bn5_b):
    N, C, H, W = r.shape
    HW = H * W
    inv_hw = 1.0 / float(HW)

    ym, xm = _sep_masks(H, W)

    wsp = jnp.concatenate([ws * inv_hw, wp], axis=0).astype(jnp.bfloat16)
    bsp = jnp.concatenate([bs * inv_hw, bp], axis=0).reshape(3 * C, 1)

    wb5 = jnp.transpose(w5, (0, 3, 2, 1))
    wb3 = jnp.zeros((C, 5, 5, C), jnp.float32)
    wb3 = wb3.at[:, 1:4, 1:4, :].set(jnp.transpose(w3, (0, 3, 2, 1)))
    wall = jnp.concatenate([wb5, wb3], axis=0)
    wc0 = wall[:, 2].reshape(2 * C, 5 * C).astype(jnp.bfloat16)
    wcs = wall[:, [0, 1, 3, 4]].reshape(2 * C, 20 * C).astype(jnp.bfloat16)

    const_args = [
        ym, xm,
        wsp, bsp,
        w1.T, b1.reshape(1, C),
        w2, we,
        g_scale.reshape(C, 1), g_bias.reshape(C, 1),
        wc0, wcs,
        bn5_s.reshape(C, 1), bn5_b.reshape(C, 1),
        bn3_s.reshape(C, 1), bn3_b.reshape(C, 1),
    ]

    img_spec = pl.BlockSpec((1, C, HW), lambda b: (b, 0, 0))

    def const_spec(a):
        idx = (0,) * a.ndim
        return pl.BlockSpec(a.shape, lambda b, _idx=idx: _idx)

    r16 = r.reshape(N, C, HW).astype(jnp.bfloat16)
    d16 = d.reshape(N, C, HW).astype(jnp.bfloat16)

    r_out, s_out, d_gl = pl.pallas_call(
        functools.partial(_gbf_body, C, H, W),
        out_shape=(jax.ShapeDtypeStruct((N, C, HW), jnp.float32),
                   jax.ShapeDtypeStruct((N, C, HW), jnp.float32),
                   jax.ShapeDtypeStruct((N, C, HW), jnp.bfloat16)),
        grid_spec=pltpu.PrefetchScalarGridSpec(
            num_scalar_prefetch=0,
            grid=(N,),
            in_specs=[img_spec, img_spec] + [const_spec(a) for a in const_args],
            out_specs=(img_spec,) * 3,
            scratch_shapes=[pltpu.VMEM((5 * C, HW), jnp.bfloat16),
                            pltpu.VMEM((20 * C, HW), jnp.bfloat16)]),
        compiler_params=pltpu.CompilerParams(dimension_semantics=("parallel",)),
    )(r16, d16, *const_args)

    By = _lowpass_matrix(H)
    Bx = _lowpass_matrix(W)
    d4 = d_gl.reshape(N, C, H, W)
    d_f = jnp.einsum('ncpw,wq->ncpq',
                     jnp.einsum('nchw,hp->ncpw', d4, By,
                                preferred_element_type=jnp.float32),
                     Bx, preferred_element_type=jnp.float32)
    r_final = r_out.reshape(N, C, H, W)
    d_final = s_out.reshape(N, C, H, W) + d_f
    return r_final, d_final


# speedup vs baseline: 1.1076x; 1.1076x over previous
import functools

import jax
import jax.numpy as jnp
import numpy as np
from jax.experimental import pallas as pl
from jax.experimental.pallas import tpu as pltpu


def _lowpass_matrix(n, cutoff_ratio=0.25):
    r = max(int(n * cutoff_ratio), 1)
    shifted = (np.arange(n) + n // 2) % n
    mask = (np.abs(shifted - n // 2) <= r).astype(np.float64)
    eye = np.eye(n)
    B = np.fft.ifft(mask[:, None] * np.fft.fft(eye, axis=0), axis=0).real
    return jnp.asarray(B, jnp.float32)


def _sep_masks(H, W):
    yy, xx = np.meshgrid(np.arange(H), np.arange(W), indexing='ij')
    ym = np.zeros((8, H * W), np.float32)
    xm = np.zeros((8, H * W), np.float32)
    for i, dlt in enumerate(range(-2, 3)):
        ym[i] = ((yy + dlt >= 0) & (yy + dlt < H)).reshape(-1)
        xm[i] = ((xx + dlt >= 0) & (xx + dlt < W)).reshape(-1)
    return (jnp.asarray(ym, jnp.bfloat16), jnp.asarray(xm, jnp.bfloat16))


def _gbf_body(C, H, W, B,
              r_ref, d_ref, ym_ref, xm_ref,
              wsp_ref, bsp_ref, w1t_ref, b1_ref, w2_ref, we_ref,
              gs_ref, gb_ref, wc0_ref, wcs_ref, s5_ref, b5_ref, s3_ref, b3_ref,
              r_out_ref, s_out_ref, d_out_ref,
              *scratches):
    HW = H * W
    wsp, bsp = wsp_ref[...], bsp_ref[...]
    w1t, b1 = w1t_ref[...], b1_ref[...]
    w2, we = w2_ref[...], we_ref[...]
    g_scale, g_bias = gs_ref[...], gb_ref[...]

    def glore(x):
        y = jnp.dot(wsp, x.astype(jnp.bfloat16),
                    preferred_element_type=jnp.float32) + bsp
        xs = y[:2 * C, :].astype(jnp.bfloat16)
        xp = y[2 * C:, :].astype(jnp.bfloat16)
        xn = jax.lax.dot_general(
            xs, xp, (((1,), (1,)), ((), ())),
            preferred_element_type=jnp.float32)
        h = jnp.dot(xn, w1t, preferred_element_type=jnp.float32) + b1
        h = jnp.maximum(h + xn, 0.0)
        xrel = jnp.dot(w2, h, preferred_element_type=jnp.float32)
        wx = jnp.dot(we, xrel, preferred_element_type=jnp.float32)
        ext = jnp.dot(wx.astype(jnp.bfloat16), xp,
                      preferred_element_type=jnp.float32)
        return x + ext * g_scale + g_bias

    rs, ss = [], []
    for i in range(B):
        r = glore(r_ref[i].astype(jnp.float32))
        d = glore(d_ref[i].astype(jnp.float32))
        s16 = (r + d).astype(jnp.bfloat16)
        dy_ref, tap_ref = scratches[2 * i], scratches[2 * i + 1]

        for k, dy in enumerate(range(-2, 3)):
            shift = (-(dy * W)) % HW
            t = pltpu.roll(s16, shift, 1) if shift else s16
            if dy:
                t = t * ym_ref[pl.ds(k, 1), :]
            dy_ref[pl.ds(k * C, C), :] = t

        stack = dy_ref[...]
        for j, dx in enumerate((-2, -1, 1, 2)):
            t = pltpu.roll(stack, (-dx) % HW, 1) * xm_ref[pl.ds(dx + 2, 1), :]
            tap_ref[pl.ds(j * 5 * C, 5 * C), :] = t

        acc = (jnp.dot(wc0_ref[...], stack, preferred_element_type=jnp.float32)
               + jnp.dot(wcs_ref[...], tap_ref[...],
                         preferred_element_type=jnp.float32))
        t5 = jnp.maximum(acc[:C] * s5_ref[...] + b5_ref[...], 0.0)
        t3 = jnp.maximum(acc[C:] * s3_ref[...] + b3_ref[...], 0.0)
        s = t5 - t3
        rs.append(s + r)
        ss.append(s)
        d_out_ref[i] = d.astype(jnp.bfloat16)

    for i in range(B):
        r_out_ref[i] = rs[i]
        s_out_ref[i] = ss[i]


def kernel(r, d, ws, bs, wp, bp, w1, b1, w2, we, g_scale, g_bias,
           w3, bn3_s, bn3_b, w5, bn5_s, bn5_b):
    N, C, H, W = r.shape
    HW = H * W
    inv_hw = 1.0 / float(HW)

    ym, xm = _sep_masks(H, W)

    wsp = jnp.concatenate([ws * inv_hw, wp], axis=0).astype(jnp.bfloat16)
    bsp = jnp.concatenate([bs * inv_hw, bp], axis=0).reshape(3 * C, 1)

    wb5 = jnp.transpose(w5, (0, 3, 2, 1))
    wb3 = jnp.zeros((C, 5, 5, C), jnp.float32)
    wb3 = wb3.at[:, 1:4, 1:4, :].set(jnp.transpose(w3, (0, 3, 2, 1)))
    wall = jnp.concatenate([wb5, wb3], axis=0)
    wc0 = wall[:, 2].reshape(2 * C, 5 * C).astype(jnp.bfloat16)
    wcs = wall[:, [0, 1, 3, 4]].reshape(2 * C, 20 * C).astype(jnp.bfloat16)

    const_args = [
        ym, xm,
        wsp, bsp,
        w1.T, b1.reshape(1, C),
        w2, we,
        g_scale.reshape(C, 1), g_bias.reshape(C, 1),
        wc0, wcs,
        bn5_s.reshape(C, 1), bn5_b.reshape(C, 1),
        bn3_s.reshape(C, 1), bn3_b.reshape(C, 1),
    ]

    B = 2
    img_spec = pl.BlockSpec((B, C, HW), lambda b: (b, 0, 0))

    def const_spec(a):
        idx = (0,) * a.ndim
        return pl.BlockSpec(a.shape, lambda b, _idx=idx: _idx)

    r_out, s_out, d_gl = pl.pallas_call(
        functools.partial(_gbf_body, C, H, W, B),
        out_shape=(jax.ShapeDtypeStruct((N, C, HW), jnp.float32),
                   jax.ShapeDtypeStruct((N, C, HW), jnp.float32),
                   jax.ShapeDtypeStruct((N, C, HW), jnp.bfloat16)),
        grid_spec=pltpu.PrefetchScalarGridSpec(
            num_scalar_prefetch=0,
            grid=(N // B,),
            in_specs=[img_spec, img_spec] + [const_spec(a) for a in const_args],
            out_specs=(img_spec,) * 3,
            scratch_shapes=[pltpu.VMEM((5 * C, HW), jnp.bfloat16),
                            pltpu.VMEM((20 * C, HW), jnp.bfloat16)] * B),
        compiler_params=pltpu.CompilerParams(dimension_semantics=("parallel",)),
    )(r.reshape(N, C, HW), d.reshape(N, C, HW), *const_args)

    By = _lowpass_matrix(H)
    Bx = _lowpass_matrix(W)
    d4 = d_gl.reshape(N, C, H, W)
    d_f = jnp.einsum('ncpw,wq->ncpq',
                     jnp.einsum('nchw,hp->ncpw', d4, By,
                                preferred_element_type=jnp.float32),
                     Bx, preferred_element_type=jnp.float32)
    r_final = r_out.reshape(N, C, H, W)
    d_final = s_out.reshape(N, C, H, W) + d_f
    return r_final, d_final


# speedup vs baseline: 1.2583x; 1.1360x over previous
import functools

import jax
import jax.numpy as jnp
import numpy as np
from jax.experimental import pallas as pl
from jax.experimental.pallas import tpu as pltpu


def _lowpass_matrix(n, cutoff_ratio=0.25):
    r = max(int(n * cutoff_ratio), 1)
    shifted = (np.arange(n) + n // 2) % n
    mask = (np.abs(shifted - n // 2) <= r).astype(np.float64)
    eye = np.eye(n)
    B = np.fft.ifft(mask[:, None] * np.fft.fft(eye, axis=0), axis=0).real
    return jnp.asarray(B, jnp.float32)


def _sep_masks(H, W):
    _, xx = np.meshgrid(np.arange(H), np.arange(W), indexing='ij')
    xm = np.zeros((8, H * W), np.float32)
    for i, dlt in enumerate(range(-2, 3)):
        xm[i] = ((xx + dlt >= 0) & (xx + dlt < W)).reshape(-1)
    return jnp.asarray(xm, jnp.bfloat16)


def _gbf_body(C, H, W, B,
              r_ref, d_ref, xm_ref,
              wsp_ref, bsp_ref, w1t_ref, b1_ref, bd_ref,
              gs_ref, gb_ref, wc0_ref, wcs_ref, s5_ref, b5_ref, s3_ref, b3_ref,
              r_out_ref, s_out_ref, d_out_ref,
              *scratches):
    HW = H * W
    wsp, bsp = wsp_ref[...], bsp_ref[...]
    w1t, b1 = w1t_ref[...], b1_ref[...]
    bd = bd_ref[...]
    g_scale, g_bias = gs_ref[...], gb_ref[...]

    def glore_front(x):
        y = jnp.dot(wsp, x.astype(jnp.bfloat16),
                    preferred_element_type=jnp.float32).astype(jnp.bfloat16)
        xs = y[:2 * C, :] + bsp[:2 * C]
        xp = y[2 * C:, :] + bsp[2 * C:]
        xn = jax.lax.dot_general(
            xs, xp, (((1,), (1,)), ((), ())),
            preferred_element_type=jnp.float32)
        return x, xp, xn

    def glore_back(x, xp, wx):
        ext = jnp.dot(wx.astype(jnp.bfloat16), xp,
                      preferred_element_type=jnp.float32)
        return x + ext * g_scale + g_bias

    fronts = [glore_front(ref[i].astype(jnp.float32))
              for i in range(B) for ref in (r_ref, d_ref)]
    xn_all = jnp.concatenate([f[2] for f in fronts], axis=0)
    h = jnp.dot(xn_all, w1t, preferred_element_type=jnp.float32) + b1
    h = jnp.maximum(h + xn_all, 0.0)
    wx_all = jnp.dot(bd, h, preferred_element_type=jnp.float32)

    outs = [glore_back(f[0], f[1], wx_all[k * C:(k + 1) * C])
            for k, f in enumerate(fronts)]

    rs, ss = [], []
    for i in range(B):
        r, d = outs[2 * i], outs[2 * i + 1]
        s16 = (r + d).astype(jnp.bfloat16)
        dx_ref, tap_ref = scratches[2 * i], scratches[2 * i + 1]

        for k, dx in enumerate(range(-2, 3)):
            t = pltpu.roll(s16, (-dx) % HW, 1) if dx else s16
            if dx:
                t = t * xm_ref[pl.ds(k, 1), :]
            dx_ref[pl.ds(k * C, C), :] = t

        stack = dx_ref[...]
        zero2 = jnp.zeros((5 * C, 2 * W), jnp.bfloat16)
        for j, dy in enumerate((-2, -1, 1, 2)):
            rows = pl.ds(j * 5 * C, 5 * C)
            if dy == 2:
                tap_ref[rows, pl.ds(0, HW - 2 * W)] = stack[:, 2 * W:]
                tap_ref[rows, pl.ds(HW - 2 * W, 2 * W)] = zero2
            elif dy == -2:
                tap_ref[rows, pl.ds(2 * W, HW - 2 * W)] = stack[:, :HW - 2 * W]
                tap_ref[rows, pl.ds(0, 2 * W)] = zero2
            elif dy == 1:
                tap_ref[rows, :] = pltpu.roll(stack, HW - W, 1)
                tap_ref[rows, pl.ds(HW - W, W)] = zero2[:, :W]
            else:
                tap_ref[rows, :] = pltpu.roll(stack, W, 1)
                tap_ref[rows, pl.ds(0, W)] = zero2[:, :W]

        acc = (jnp.dot(wc0_ref[...], stack, preferred_element_type=jnp.float32)
               + jnp.dot(wcs_ref[...], tap_ref[...],
                         preferred_element_type=jnp.float32))
        t5 = jnp.maximum(acc[:C] * s5_ref[...] + b5_ref[...], 0.0)
        t3 = jnp.maximum(acc[C:] * s3_ref[...] + b3_ref[...], 0.0)
        s = t5 - t3
        rs.append(s + r)
        ss.append(s)
        d_out_ref[i] = d.astype(jnp.bfloat16)

    for i in range(B):
        r_out_ref[i] = rs[i]
        s_out_ref[i] = ss[i]


def kernel(r, d, ws, bs, wp, bp, w1, b1, w2, we, g_scale, g_bias,
           w3, bn3_s, bn3_b, w5, bn5_s, bn5_b):
    N, C, H, W = r.shape
    HW = H * W
    inv_hw = 1.0 / float(HW)

    xm = _sep_masks(H, W)

    wsp = jnp.concatenate([ws * inv_hw, wp], axis=0).astype(jnp.bfloat16)
    bsp = (jnp.concatenate([bs * inv_hw, bp], axis=0)
           .reshape(3 * C, 1).astype(jnp.bfloat16))

    wb5 = jnp.transpose(w5, (0, 2, 3, 1))
    wb3 = jnp.zeros((C, 5, 5, C), jnp.float32)
    wb3 = wb3.at[:, 1:4, 1:4, :].set(jnp.transpose(w3, (0, 2, 3, 1)))
    wall = jnp.concatenate([wb5, wb3], axis=0)
    wc0 = wall[:, 2].reshape(2 * C, 5 * C).astype(jnp.bfloat16)
    wcs = wall[:, [0, 1, 3, 4]].reshape(2 * C, 20 * C).astype(jnp.bfloat16)

    ww2 = jnp.kron(jnp.eye(4, dtype=jnp.float32), jnp.dot(we, w2))

    const_args = [
        xm,
        wsp, bsp,
        w1.T, b1.reshape(1, C),
        ww2,
        g_scale.reshape(C, 1), g_bias.reshape(C, 1),
        wc0, wcs,
        bn5_s.reshape(C, 1), bn5_b.reshape(C, 1),
        bn3_s.reshape(C, 1), bn3_b.reshape(C, 1),
    ]

    B = 2
    img_spec = pl.BlockSpec((B, C, HW), lambda b: (b, 0, 0))

    def const_spec(a):
        idx = (0,) * a.ndim
        return pl.BlockSpec(a.shape, lambda b, _idx=idx: _idx)

    r_out, s_out, d_gl = pl.pallas_call(
        functools.partial(_gbf_body, C, H, W, B),
        out_shape=(jax.ShapeDtypeStruct((N, C, HW), jnp.float32),
                   jax.ShapeDtypeStruct((N, C, HW), jnp.float32),
                   jax.ShapeDtypeStruct((N, C, HW), jnp.bfloat16)),
        grid_spec=pltpu.PrefetchScalarGridSpec(
            num_scalar_prefetch=0,
            grid=(N // B,),
            in_specs=[img_spec, img_spec] + [const_spec(a) for a in const_args],
            out_specs=(img_spec,) * 3,
            scratch_shapes=[pltpu.VMEM((5 * C, HW), jnp.bfloat16),
                            pltpu.VMEM((20 * C, HW), jnp.bfloat16)] * B),
        compiler_params=pltpu.CompilerParams(dimension_semantics=("parallel",)),
    )(r.reshape(N, C, HW), d.reshape(N, C, HW), *const_args)

    By = _lowpass_matrix(H)
    Bx = _lowpass_matrix(W)
    d4 = d_gl.reshape(N, C, H, W)
    d_f = jnp.einsum('ncpw,wq->ncpq',
                     jnp.einsum('nchw,hp->ncpw', d4, By,
                                preferred_element_type=jnp.float32),
                     Bx, preferred_element_type=jnp.float32)
    r_final = r_out.reshape(N, C, H, W)
    d_final = s_out.reshape(N, C, H, W) + d_f
    return r_final, d_final


# speedup vs baseline: 1.2885x; 1.0240x over previous
import functools

import jax
import jax.numpy as jnp
import numpy as np
from jax.experimental import pallas as pl
from jax.experimental.pallas import tpu as pltpu


def _lowpass_matrix(n, cutoff_ratio=0.25):
    r = max(int(n * cutoff_ratio), 1)
    shifted = (np.arange(n) + n // 2) % n
    mask = (np.abs(shifted - n // 2) <= r).astype(np.float64)
    eye = np.eye(n)
    B = np.fft.ifft(mask[:, None] * np.fft.fft(eye, axis=0), axis=0).real
    return jnp.asarray(B, jnp.float32)


def _sep_masks(H, W):
    _, xx = np.meshgrid(np.arange(H), np.arange(W), indexing='ij')
    xm = np.zeros((8, H * W), np.float32)
    for i, dlt in enumerate(range(-2, 3)):
        xm[i] = ((xx + dlt >= 0) & (xx + dlt < W)).reshape(-1)
    return jnp.asarray(xm, jnp.bfloat16)


def _gbf_body(C, H, W, B,
              r_ref, d_ref, xm_ref,
              wsp_ref, bsp_ref, w1t_ref, b1_ref, bd_ref,
              gs_ref, gb_ref, wc0_ref, wcs_ref, s5_ref, b5_ref, s3_ref, b3_ref,
              r_out_ref, s_out_ref, d_out_ref,
              *scratches):
    HW = H * W
    wsp, bsp = wsp_ref[...], bsp_ref[...]
    w1t, b1 = w1t_ref[...], b1_ref[...]
    bd = bd_ref[...]
    g_scale, g_bias = gs_ref[...], gb_ref[...]

    def glore_front(x):
        y = jnp.dot(wsp, x.astype(jnp.bfloat16),
                    preferred_element_type=jnp.float32).astype(jnp.bfloat16)
        xs = y[:2 * C, :] + bsp[:2 * C]
        xp = y[2 * C:, :] + bsp[2 * C:]
        xn = jax.lax.dot_general(
            xs, xp, (((1,), (1,)), ((), ())),
            preferred_element_type=jnp.float32)
        return x, xp, xn

    def glore_back(x, xp, wx):
        ext = jnp.dot(wx.astype(jnp.bfloat16), xp,
                      preferred_element_type=jnp.float32)
        return x + ext * g_scale + g_bias

    fronts = [glore_front(ref[i].astype(jnp.float32))
              for i in range(B) for ref in (r_ref, d_ref)]
    xn_all = jnp.concatenate([f[2] for f in fronts], axis=0)
    h = jnp.dot(xn_all, w1t, preferred_element_type=jnp.float32) + b1
    h = jnp.maximum(h + xn_all, 0.0)
    wx_all = jnp.dot(bd, h, preferred_element_type=jnp.float32)

    outs = [glore_back(f[0], f[1], wx_all[k * C:(k + 1) * C])
            for k, f in enumerate(fronts)]

    rs, ss = [], []
    for i in range(B):
        r, d = outs[2 * i], outs[2 * i + 1]
        s16 = (r + d).astype(jnp.bfloat16)
        dx_ref, tap_ref = scratches[2 * i], scratches[2 * i + 1]

        for k, dx in enumerate(range(-2, 3)):
            t = pltpu.roll(s16, (-dx) % HW, 1) if dx else s16
            if dx:
                t = t * xm_ref[pl.ds(k, 1), :]
            dx_ref[pl.ds(k * C, C), :] = t

        stack = dx_ref[...]
        zero2 = jnp.zeros((5 * C, 2 * W), jnp.bfloat16)
        for j, dy in enumerate((-2, -1, 1, 2)):
            rows = pl.ds(j * 5 * C, 5 * C)
            if dy == 2:
                tap_ref[rows, pl.ds(0, HW - 2 * W)] = stack[:, 2 * W:]
                tap_ref[rows, pl.ds(HW - 2 * W, 2 * W)] = zero2
            elif dy == -2:
                tap_ref[rows, pl.ds(2 * W, HW - 2 * W)] = stack[:, :HW - 2 * W]
                tap_ref[rows, pl.ds(0, 2 * W)] = zero2
            elif dy == 1:
                tap_ref[rows, :] = pltpu.roll(stack, HW - W, 1)
                tap_ref[rows, pl.ds(HW - W, W)] = zero2[:, :W]
            else:
                tap_ref[rows, :] = pltpu.roll(stack, W, 1)
                tap_ref[rows, pl.ds(0, W)] = zero2[:, :W]

        acc = (jnp.dot(wc0_ref[...], stack, preferred_element_type=jnp.float32)
               + jnp.dot(wcs_ref[...], tap_ref[...],
                         preferred_element_type=jnp.float32))
        t5 = jnp.maximum(acc[:C] * s5_ref[...] + b5_ref[...], 0.0)
        t3 = jnp.maximum(acc[C:] * s3_ref[...] + b3_ref[...], 0.0)
        s = t5 - t3
        rs.append(s + r)
        ss.append(s)
        d_out_ref[i] = d.astype(jnp.bfloat16)

    for i in range(B):
        r_out_ref[i] = rs[i]
        s_out_ref[i] = ss[i]


def kernel(r, d, ws, bs, wp, bp, w1, b1, w2, we, g_scale, g_bias,
           w3, bn3_s, bn3_b, w5, bn5_s, bn5_b):
    N, C, H, W = r.shape
    HW = H * W
    inv_hw = 1.0 / float(HW)

    xm = _sep_masks(H, W)

    wsp = jnp.concatenate([ws * inv_hw, wp], axis=0).astype(jnp.bfloat16)
    bsp = (jnp.concatenate([bs * inv_hw, bp], axis=0)
           .reshape(3 * C, 1).astype(jnp.bfloat16))

    wb5 = jnp.transpose(w5, (0, 2, 3, 1))
    wb3 = jnp.zeros((C, 5, 5, C), jnp.float32)
    wb3 = wb3.at[:, 1:4, 1:4, :].set(jnp.transpose(w3, (0, 2, 3, 1)))
    wall = jnp.concatenate([wb5, wb3], axis=0)
    wc0 = wall[:, 2].reshape(2 * C, 5 * C).astype(jnp.bfloat16)
    wcs = wall[:, [0, 1, 3, 4]].reshape(2 * C, 20 * C).astype(jnp.bfloat16)

    ww2 = jnp.kron(jnp.eye(8, dtype=jnp.float32), jnp.dot(we, w2))

    const_args = [
        xm,
        wsp, bsp,
        w1.T, b1.reshape(1, C),
        ww2,
        g_scale.reshape(C, 1), g_bias.reshape(C, 1),
        wc0, wcs,
        bn5_s.reshape(C, 1), bn5_b.reshape(C, 1),
        bn3_s.reshape(C, 1), bn3_b.reshape(C, 1),
    ]

    B = 4
    img_spec = pl.BlockSpec((B, C, HW), lambda b: (b, 0, 0))

    def const_spec(a):
        idx = (0,) * a.ndim
        return pl.BlockSpec(a.shape, lambda b, _idx=idx: _idx)

    r_out, s_out, d_gl = pl.pallas_call(
        functools.partial(_gbf_body, C, H, W, B),
        out_shape=(jax.ShapeDtypeStruct((N, C, HW), jnp.float32),
                   jax.ShapeDtypeStruct((N, C, HW), jnp.float32),
                   jax.ShapeDtypeStruct((N, C, HW), jnp.bfloat16)),
        grid_spec=pltpu.PrefetchScalarGridSpec(
            num_scalar_prefetch=0,
            grid=(N // B,),
            in_specs=[img_spec, img_spec] + [const_spec(a) for a in const_args],
            out_specs=(img_spec,) * 3,
            scratch_shapes=[pltpu.VMEM((5 * C, HW), jnp.bfloat16),
                            pltpu.VMEM((20 * C, HW), jnp.bfloat16)] * B),
        compiler_params=pltpu.CompilerParams(dimension_semantics=("parallel",)),
    )(r.reshape(N, C, HW), d.reshape(N, C, HW), *const_args)

    By = _lowpass_matrix(H)
    Bx = _lowpass_matrix(W)
    d4 = d_gl.reshape(N, C, H, W)
    d_f = jnp.einsum('ncpw,wq->ncpq',
                     jnp.einsum('nchw,hp->ncpw', d4, By,
                                preferred_element_type=jnp.float32),
                     Bx, preferred_element_type=jnp.float32)
    r_final = r_out.reshape(N, C, H, W)
    d_final = s_out.reshape(N, C, H, W) + d_f
    return r_final, d_final


# speedup vs baseline: 1.3061x; 1.0137x over previous
import functools

import jax
import jax.numpy as jnp
import numpy as np
from jax.experimental import pallas as pl
from jax.experimental.pallas import tpu as pltpu


def _lowpass_matrix(n, cutoff_ratio=0.25):
    r = max(int(n * cutoff_ratio), 1)
    shifted = (np.arange(n) + n // 2) % n
    mask = (np.abs(shifted - n // 2) <= r).astype(np.float64)
    eye = np.eye(n)
    B = np.fft.ifft(mask[:, None] * np.fft.fft(eye, axis=0), axis=0).real
    return jnp.asarray(B, jnp.float32)


def _sep_masks(H, W):
    _, xx = np.meshgrid(np.arange(H), np.arange(W), indexing='ij')
    xm = np.zeros((8, H * W), np.float32)
    for i, dlt in enumerate(range(-2, 3)):
        xm[i] = ((xx + dlt >= 0) & (xx + dlt < W)).reshape(-1)
    return jnp.asarray(xm, jnp.bfloat16)


def _gbf_body(C, H, W, B,
              r_ref, d_ref, xm_ref,
              wsp_ref, bsp_ref, w1t_ref, b1_ref, bd_ref,
              gs_ref, gb_ref, wc0_ref, wcs_ref, s5_ref, b5_ref, s3_ref, b3_ref,
              r_out_ref, s_out_ref, d_out_ref,
              *scratches):
    HW = H * W
    wsp, bsp = wsp_ref[...], bsp_ref[...]
    w1t, b1 = w1t_ref[...], b1_ref[...]
    bd = bd_ref[...]
    g_scale, g_bias = gs_ref[...], gb_ref[...]

    def glore_front(x):
        y = jnp.dot(wsp, x.astype(jnp.bfloat16),
                    preferred_element_type=jnp.float32).astype(jnp.bfloat16)
        xs = y[:2 * C, :] + bsp[:2 * C]
        xp = y[2 * C:, :] + bsp[2 * C:]
        xn = jax.lax.dot_general(
            xs, xp, (((1,), (1,)), ((), ())),
            preferred_element_type=jnp.float32)
        return x, xp, xn

    def glore_back(x, xp, wx):
        ext = jnp.dot(wx.astype(jnp.bfloat16), xp,
                      preferred_element_type=jnp.float32)
        return x + ext * g_scale + g_bias

    fronts = [glore_front(ref[i].astype(jnp.float32))
              for i in range(B) for ref in (r_ref, d_ref)]
    xn_all = jnp.concatenate([f[2] for f in fronts], axis=0)
    h = jnp.dot(xn_all, w1t, preferred_element_type=jnp.float32) + b1
    h = jnp.maximum(h + xn_all, 0.0)
    wx_all = jnp.dot(bd, h, preferred_element_type=jnp.float32)

    outs = [glore_back(f[0], f[1], wx_all[k * C:(k + 1) * C])
            for k, f in enumerate(fronts)]

    rs, ss = [], []
    for i in range(B):
        r, d = outs[2 * i], outs[2 * i + 1]
        s16 = (r + d).astype(jnp.bfloat16)
        dx_ref, tap_ref = scratches[2 * i], scratches[2 * i + 1]

        for k, dx in enumerate(range(-2, 3)):
            t = pltpu.roll(s16, (-dx) % HW, 1) if dx else s16
            if dx:
                t = t * xm_ref[pl.ds(k, 1), :]
            dx_ref[pl.ds(k * C, C), :] = t

        stack = dx_ref[...]
        zero2 = jnp.zeros((5 * C, 2 * W), jnp.bfloat16)
        for j, dy in enumerate((-2, -1, 1, 2)):
            rows = pl.ds(j * 5 * C, 5 * C)
            if dy == 2:
                tap_ref[rows, pl.ds(0, HW - 2 * W)] = stack[:, 2 * W:]
                tap_ref[rows, pl.ds(HW - 2 * W, 2 * W)] = zero2
            elif dy == -2:
                tap_ref[rows, pl.ds(2 * W, HW - 2 * W)] = stack[:, :HW - 2 * W]
                tap_ref[rows, pl.ds(0, 2 * W)] = zero2
            elif dy == 1:
                tap_ref[rows, :] = pltpu.roll(stack, HW - W, 1)
                tap_ref[rows, pl.ds(HW - W, W)] = zero2[:, :W]
            else:
                tap_ref[rows, :] = pltpu.roll(stack, W, 1)
                tap_ref[rows, pl.ds(0, W)] = zero2[:, :W]

        acc = (jnp.dot(wc0_ref[...], stack, preferred_element_type=jnp.float32)
               + jnp.dot(wcs_ref[...], tap_ref[...],
                         preferred_element_type=jnp.float32))
        t5 = jnp.maximum(acc[:C] * s5_ref[...] + b5_ref[...], 0.0)
        t3 = jnp.maximum(acc[C:] * s3_ref[...] + b3_ref[...], 0.0)
        s = t5 - t3
        rs.append(s + r)
        ss.append(s)
        d_out_ref[i] = d.astype(jnp.bfloat16)

    for i in range(B):
        r_out_ref[i] = rs[i]
        s_out_ref[i] = ss[i].astype(jnp.bfloat16)


def kernel(r, d, ws, bs, wp, bp, w1, b1, w2, we, g_scale, g_bias,
           w3, bn3_s, bn3_b, w5, bn5_s, bn5_b):
    N, C, H, W = r.shape
    HW = H * W
    inv_hw = 1.0 / float(HW)

    xm = _sep_masks(H, W)

    wsp = jnp.concatenate([ws * inv_hw, wp], axis=0).astype(jnp.bfloat16)
    bsp = (jnp.concatenate([bs * inv_hw, bp], axis=0)
           .reshape(3 * C, 1).astype(jnp.bfloat16))

    wb5 = jnp.transpose(w5, (0, 2, 3, 1))
    wb3 = jnp.zeros((C, 5, 5, C), jnp.float32)
    wb3 = wb3.at[:, 1:4, 1:4, :].set(jnp.transpose(w3, (0, 2, 3, 1)))
    wall = jnp.concatenate([wb5, wb3], axis=0)
    wc0 = wall[:, 2].reshape(2 * C, 5 * C).astype(jnp.bfloat16)
    wcs = wall[:, [0, 1, 3, 4]].reshape(2 * C, 20 * C).astype(jnp.bfloat16)

    B = next(b for b in (4, 2, 1) if N % b == 0)
    ww2 = jnp.kron(jnp.eye(2 * B, dtype=jnp.float32), jnp.dot(we, w2))

    const_args = [
        xm,
        wsp, bsp,
        w1.T, b1.reshape(1, C),
        ww2,
        g_scale.reshape(C, 1), g_bias.reshape(C, 1),
        wc0, wcs,
        bn5_s.reshape(C, 1), bn5_b.reshape(C, 1),
        bn3_s.reshape(C, 1), bn3_b.reshape(C, 1),
    ]

    img_spec = pl.BlockSpec((B, C, HW), lambda b: (b, 0, 0))

    def const_spec(a):
        idx = (0,) * a.ndim
        return pl.BlockSpec(a.shape, lambda b, _idx=idx: _idx)

    r_out, s_out, d_gl = pl.pallas_call(
        functools.partial(_gbf_body, C, H, W, B),
        out_shape=(jax.ShapeDtypeStruct((N, C, HW), jnp.float32),
                   jax.ShapeDtypeStruct((N, C, HW), jnp.bfloat16),
                   jax.ShapeDtypeStruct((N, C, HW), jnp.bfloat16)),
        grid_spec=pltpu.PrefetchScalarGridSpec(
            num_scalar_prefetch=0,
            grid=(N // B,),
            in_specs=[img_spec, img_spec] + [const_spec(a) for a in const_args],
            out_specs=(img_spec,) * 3,
            scratch_shapes=[pltpu.VMEM((5 * C, HW), jnp.bfloat16),
                            pltpu.VMEM((20 * C, HW), jnp.bfloat16)] * B),
        compiler_params=pltpu.CompilerParams(dimension_semantics=("parallel",)),
    )(r.reshape(N, C, HW), d.reshape(N, C, HW), *const_args)

    By = _lowpass_matrix(H)
    Bx = _lowpass_matrix(W)
    d4 = d_gl.reshape(N, C, H, W)
    d_f = jnp.einsum('ncpw,wq->ncpq',
                     jnp.einsum('nchw,hp->ncpw', d4, By,
                                preferred_element_type=jnp.float32),
                     Bx, preferred_element_type=jnp.float32)
    r_final = r_out.reshape(N, C, H, W)
    d_final = s_out.reshape(N, C, H, W) + d_f
    return r_final, d_final


# speedup vs baseline: 1.3076x; 1.0011x over previous
import functools

import jax
import jax.numpy as jnp
import numpy as np
from jax.experimental import pallas as pl
from jax.experimental.pallas import tpu as pltpu


def _lowpass_matrix(n, cutoff_ratio=0.25):
    r = max(int(n * cutoff_ratio), 1)
    shifted = (np.arange(n) + n // 2) % n
    mask = (np.abs(shifted - n // 2) <= r).astype(np.float64)
    eye = np.eye(n)
    B = np.fft.ifft(mask[:, None] * np.fft.fft(eye, axis=0), axis=0).real
    return jnp.asarray(B, jnp.float32)


def _sep_masks(H, W):
    _, xx = np.meshgrid(np.arange(H), np.arange(W), indexing='ij')
    xm = np.zeros((8, H * W), np.float32)
    for i, dlt in enumerate(range(-2, 3)):
        xm[i] = ((xx + dlt >= 0) & (xx + dlt < W)).reshape(-1)
    return jnp.asarray(xm, jnp.bfloat16)


def _gbf_body(C, H, W, B,
              r_ref, d_ref, xm_ref,
              wsp_ref, bsp_ref, w1t_ref, b1_ref, bd_ref,
              gs_ref, gb_ref, wc0_ref, wcs_ref, s5_ref, b5_ref, s3_ref, b3_ref,
              r_out_ref, s_out_ref, d_out_ref,
              *scratches):
    HW = H * W
    wsp, bsp = wsp_ref[...], bsp_ref[...]
    w1t, b1 = w1t_ref[...], b1_ref[...]
    bd = bd_ref[...]
    g_scale, g_bias = gs_ref[...], gb_ref[...]

    def glore_front(x):
        y = jnp.dot(wsp, x.astype(jnp.bfloat16),
                    preferred_element_type=jnp.float32).astype(jnp.bfloat16)
        xs = y[:2 * C, :] + bsp[:2 * C]
        xp = y[2 * C:, :] + bsp[2 * C:]
        xn = jax.lax.dot_general(
            xs, xp, (((1,), (1,)), ((), ())),
            preferred_element_type=jnp.float32)
        return x, xp, xn

    def glore_back(x, xp, wx):
        ext = jnp.dot(wx.astype(jnp.bfloat16), xp,
                      preferred_element_type=jnp.float32)
        return x + ext * g_scale + g_bias

    fronts = [glore_front(ref[i].astype(jnp.float32))
              for i in range(B) for ref in (r_ref, d_ref)]
    xn_all = jnp.concatenate([f[2] for f in fronts], axis=0)
    h = jnp.dot(xn_all, w1t, preferred_element_type=jnp.float32) + b1
    h = jnp.maximum(h + xn_all, 0.0)
    wx_all = jnp.dot(bd, h, preferred_element_type=jnp.float32)

    outs = [glore_back(f[0], f[1], wx_all[k * C:(k + 1) * C])
            for k, f in enumerate(fronts)]

    for i in range(B):
        r, d = outs[2 * i], outs[2 * i + 1]
        s16 = (r + d).astype(jnp.bfloat16)
        dx_ref, tap_ref = scratches[2 * i], scratches[2 * i + 1]

        for k, dx in enumerate(range(-2, 3)):
            t = pltpu.roll(s16, (-dx) % HW, 1) if dx else s16
            if dx:
                t = t * xm_ref[pl.ds(k, 1), :]
            dx_ref[pl.ds(k * C, C), :] = t

        stack = dx_ref[...]
        zero2 = jnp.zeros((5 * C, 2 * W), jnp.bfloat16)
        for j, dy in enumerate((-2, -1, 1, 2)):
            rows = pl.ds(j * 5 * C, 5 * C)
            if dy == 2:
                tap_ref[rows, pl.ds(0, HW - 2 * W)] = stack[:, 2 * W:]
                tap_ref[rows, pl.ds(HW - 2 * W, 2 * W)] = zero2
            elif dy == -2:
                tap_ref[rows, pl.ds(2 * W, HW - 2 * W)] = stack[:, :HW - 2 * W]
                tap_ref[rows, pl.ds(0, 2 * W)] = zero2
            elif dy == 1:
                tap_ref[rows, :] = pltpu.roll(stack, HW - W, 1)
                tap_ref[rows, pl.ds(HW - W, W)] = zero2[:, :W]
            else:
                tap_ref[rows, :] = pltpu.roll(stack, W, 1)
                tap_ref[rows, pl.ds(0, W)] = zero2[:, :W]

        acc = (jnp.dot(wc0_ref[...], stack, preferred_element_type=jnp.float32)
               + jnp.dot(wcs_ref[...], tap_ref[...],
                         preferred_element_type=jnp.float32))
        t5 = jnp.maximum(acc[:C] * s5_ref[...] + b5_ref[...], 0.0)
        t3 = jnp.maximum(acc[C:] * s3_ref[...] + b3_ref[...], 0.0)
        s = t5 - t3
        r_out_ref[i] = s + r
        s_out_ref[i] = s.astype(jnp.bfloat16)
        d_out_ref[i] = d.astype(jnp.bfloat16)


def kernel(r, d, ws, bs, wp, bp, w1, b1, w2, we, g_scale, g_bias,
           w3, bn3_s, bn3_b, w5, bn5_s, bn5_b):
    N, C, H, W = r.shape
    HW = H * W
    inv_hw = 1.0 / float(HW)

    xm = _sep_masks(H, W)

    wsp = jnp.concatenate([ws * inv_hw, wp], axis=0).astype(jnp.bfloat16)
    bsp = (jnp.concatenate([bs * inv_hw, bp], axis=0)
           .reshape(3 * C, 1).astype(jnp.bfloat16))

    wb5 = jnp.transpose(w5, (0, 2, 3, 1))
    wb3 = jnp.zeros((C, 5, 5, C), jnp.float32)
    wb3 = wb3.at[:, 1:4, 1:4, :].set(jnp.transpose(w3, (0, 2, 3, 1)))
    wall = jnp.concatenate([wb5, wb3], axis=0)
    wc0 = wall[:, 2].reshape(2 * C, 5 * C).astype(jnp.bfloat16)
    wcs = wall[:, [0, 1, 3, 4]].reshape(2 * C, 20 * C).astype(jnp.bfloat16)

    B = next(b for b in (4, 2, 1) if N % b == 0)
    ww2 = jnp.kron(jnp.eye(2 * B, dtype=jnp.float32), jnp.dot(we, w2))

    const_args = [
        xm,
        wsp, bsp,
        w1.T, b1.reshape(1, C),
        ww2,
        g_scale.reshape(C, 1), g_bias.reshape(C, 1),
        wc0, wcs,
        bn5_s.reshape(C, 1), bn5_b.reshape(C, 1),
        bn3_s.reshape(C, 1), bn3_b.reshape(C, 1),
    ]

    img_spec = pl.BlockSpec((B, C, HW), lambda b: (b, 0, 0))

    def const_spec(a):
        idx = (0,) * a.ndim
        return pl.BlockSpec(a.shape, lambda b, _idx=idx: _idx)

    r_out, s_out, d_gl = pl.pallas_call(
        functools.partial(_gbf_body, C, H, W, B),
        out_shape=(jax.ShapeDtypeStruct((N, C, HW), jnp.float32),
                   jax.ShapeDtypeStruct((N, C, HW), jnp.bfloat16),
                   jax.ShapeDtypeStruct((N, C, HW), jnp.bfloat16)),
        grid_spec=pltpu.PrefetchScalarGridSpec(
            num_scalar_prefetch=0,
            grid=(N // B,),
            in_specs=[img_spec, img_spec] + [const_spec(a) for a in const_args],
            out_specs=(img_spec,) * 3,
            scratch_shapes=[pltpu.VMEM((5 * C, HW), jnp.bfloat16),
                            pltpu.VMEM((20 * C, HW), jnp.bfloat16)] * B),
        compiler_params=pltpu.CompilerParams(dimension_semantics=("parallel",)),
    )(r.reshape(N, C, HW), d.reshape(N, C, HW), *const_args)

    By = _lowpass_matrix(H)
    Bx = _lowpass_matrix(W)
    d4 = d_gl.reshape(N, C, H, W)
    d_f = jnp.einsum('ncpw,wq->ncpq',
                     jnp.einsum('nchw,hp->ncpw', d4, By,
                                preferred_element_type=jnp.float32),
                     Bx, preferred_element_type=jnp.float32)
    r_final = r_out.reshape(N, C, H, W)
    d_final = s_out.reshape(N, C, H, W) + d_f
    return r_final, d_final
